# baseline (speedup 1.0000x reference)
import jax
import jax.numpy as jnp
from jax.experimental import pallas as pl
from jax.experimental.pallas import tpu as pltpu
from jax.experimental.pallas import tpu_sc as plsc

_BM = 512
_KT = 2048
_BL = 2048


def _sim_argmax_body(z_ref, cb_ref, ids_ref, cbn_ref):
    i = pl.program_id(0)

    @pl.when(i == 0)
    def _normalize_codebook():
        cb = cb_ref[...]
        nrm = jnp.sqrt(jnp.sum(cb * cb, axis=1, keepdims=True))
        cbn_ref[...] = cb / jnp.maximum(nrm, 1e-12)

    z = z_ref[...]
    zn = z / jnp.maximum(jnp.sqrt(jnp.sum(z * z, axis=1, keepdims=True)), 1e-12)
    bm = z.shape[0]
    k = cb_ref.shape[0]
    kt = _KT
    run_max = jnp.full((bm, 128), -jnp.inf, jnp.float32)
    run_blk = jnp.zeros((bm, 128), jnp.int32)
    for t in range(k // kt):
        s = jax.lax.dot_general(
            zn, cbn_ref[pl.ds(t * kt, kt), :],
            dimension_numbers=(((1,), (1,)), ((), ())),
            preferred_element_type=jnp.float32,
        )
        for sub in range(kt // 128):
            x = s[:, sub * 128:(sub + 1) * 128]
            gt = x > run_max
            run_max = jnp.where(gt, x, run_max)
            run_blk = jnp.where(gt, t * (kt // 128) + sub, run_blk)
    j = jax.lax.broadcasted_iota(jnp.int32, (bm, 128), 1)
    kfull = run_blk * 128 + j
    maxv = jnp.max(run_max, axis=1, keepdims=True)
    ids_ref[0, 0, :] = jnp.min(jnp.where(run_max == maxv, kfull, k), axis=1)


def _loss_body(z_ref, zq_ref, out_ref):
    d = z_ref[...] - zq_ref[...]
    out_ref[...] = jnp.sum(d * d).reshape(1, 1, 1)


def _gather_rows(codebook, ids2d, n_rows, dim):
    mesh = plsc.VectorSubcoreMesh(core_axis_name="core", subcore_axis_name="subcore")
    window = 128

    @pl.kernel(
        out_type=jax.ShapeDtypeStruct((n_rows, dim), codebook.dtype),
        mesh=mesh,
    )
    def gather_kernel(cb_hbm, i_hbm, o_hbm):
        def body(i_vmem, o_vmem):
            pltpu.sync_copy(cb_hbm.at[i_vmem.at[0]], o_vmem)

        pltpu.emit_pipeline(
            body,
            grid=(n_rows // window,),
            in_specs=[pl.BlockSpec((1, window), lambda i: (0, i))],
            out_specs=[pl.BlockSpec((window, dim), lambda i: (i, 0))],
            core_axis_name=("core", "subcore"),
            dimension_semantics=(pltpu.PARALLEL,),
        )(i_hbm, o_hbm)

    return gather_kernel(codebook, ids2d)


def kernel(z, codebook):
    b, d = z.shape
    k, _ = codebook.shape
    nb = b // _BM

    ids3 = pl.pallas_call(
        _sim_argmax_body,
        grid=(nb,),
        in_specs=[
            pl.BlockSpec((_BM, d), lambda i: (i, 0)),
            pl.BlockSpec((k, d), lambda i: (0, 0)),
        ],
        out_specs=pl.BlockSpec((1, 1, _BM), lambda i: (i, 0, 0)),
        out_shape=jax.ShapeDtypeStruct((nb, 1, _BM), jnp.int32),
        scratch_shapes=[pltpu.VMEM((k, d), jnp.float32)],
        compiler_params=pltpu.CompilerParams(
            dimension_semantics=(pltpu.ARBITRARY,),
        ),
    )(z, codebook)
    code_ids = ids3.reshape(b)

    z_q = _gather_rows(codebook, ids3.reshape(1, b), b, d)

    nl = b // _BL
    partials = pl.pallas_call(
        _loss_body,
        grid=(nl,),
        in_specs=[
            pl.BlockSpec((_BL, d), lambda i: (i, 0)),
            pl.BlockSpec((_BL, d), lambda i: (i, 0)),
        ],
        out_specs=pl.BlockSpec((1, 1, 1), lambda i: (i, 0, 0)),
        out_shape=jax.ShapeDtypeStruct((nl, 1, 1), jnp.float32),
        compiler_params=pltpu.CompilerParams(
            dimension_semantics=(pltpu.PARALLEL,),
        ),
    )(z, z_q)
    loss = (jnp.sum(partials) * (0.25 / (b * d))).astype(jnp.float32)

    return (z_q, code_ids, loss)


# speedup vs baseline: 1.0089x; 1.0089x over previous
import jax
import jax.numpy as jnp
from jax.experimental import pallas as pl
from jax.experimental.pallas import tpu as pltpu
from jax.experimental.pallas import tpu_sc as plsc

_BM = 512
_KT = 1024
_BL = 4096


def _sim_argmax_body(z_ref, cb_ref, ids_ref, cbn_ref):
    i = pl.program_id(0)

    @pl.when(i == 0)
    def _normalize_codebook():
        cb = cb_ref[...]
        nrm = jnp.sqrt(jnp.sum(cb * cb, axis=1, keepdims=True))
        cbn_ref[...] = cb / jnp.maximum(nrm, 1e-12)

    z = z_ref[...]
    zn = z / jnp.maximum(jnp.sqrt(jnp.sum(z * z, axis=1, keepdims=True)), 1e-12)
    bm = z.shape[0]
    k = cb_ref.shape[0]
    kt = _KT
    run_max = jnp.full((bm, 128), -jnp.inf, jnp.float32)
    run_blk = jnp.zeros((bm, 128), jnp.int32)
    for t in range(k // kt):
        s = jax.lax.dot_general(
            zn, cbn_ref[pl.ds(t * kt, kt), :],
            dimension_numbers=(((1,), (1,)), ((), ())),
            preferred_element_type=jnp.float32,
        )
        for sub in range(kt // 128):
            x = s[:, sub * 128:(sub + 1) * 128]
            gt = x > run_max
            run_max = jnp.where(gt, x, run_max)
            run_blk = jnp.where(gt, t * (kt // 128) + sub, run_blk)
    j = jax.lax.broadcasted_iota(jnp.int32, (bm, 128), 1)
    kfull = run_blk * 128 + j
    maxv = jnp.max(run_max, axis=1, keepdims=True)
    ids_ref[0, 0, :] = jnp.min(jnp.where(run_max == maxv, kfull, k), axis=1)


def _loss_body(z_ref, zq_ref, out_ref):
    d = z_ref[...] - zq_ref[...]
    out_ref[...] = jnp.sum(d * d).reshape(1, 1, 1)


def _gather_rows(codebook, ids2d, n_rows, dim):
    mesh = plsc.VectorSubcoreMesh(core_axis_name="core", subcore_axis_name="subcore")
    window = 128

    @pl.kernel(
        out_type=jax.ShapeDtypeStruct((n_rows, dim), codebook.dtype),
        mesh=mesh,
    )
    def gather_kernel(cb_hbm, i_hbm, o_hbm):
        def body(i_vmem, o_vmem):
            pltpu.sync_copy(cb_hbm.at[i_vmem.at[0]], o_vmem)

        pltpu.emit_pipeline(
            body,
            grid=(n_rows // window,),
            in_specs=[pl.BlockSpec((1, window), lambda i: (0, i))],
            out_specs=[pl.BlockSpec((window, dim), lambda i: (i, 0))],
            core_axis_name=("core", "subcore"),
            dimension_semantics=(pltpu.PARALLEL,),
        )(i_hbm, o_hbm)

    return gather_kernel(codebook, ids2d)


def kernel(z, codebook):
    b, d = z.shape
    k, _ = codebook.shape
    nb = b // _BM

    ids3 = pl.pallas_call(
        _sim_argmax_body,
        grid=(nb,),
        in_specs=[
            pl.BlockSpec((_BM, d), lambda i: (i, 0)),
            pl.BlockSpec((k, d), lambda i: (0, 0)),
        ],
        out_specs=pl.BlockSpec((1, 1, _BM), lambda i: (i, 0, 0)),
        out_shape=jax.ShapeDtypeStruct((nb, 1, _BM), jnp.int32),
        scratch_shapes=[pltpu.VMEM((k, d), jnp.float32)],
        compiler_params=pltpu.CompilerParams(
            dimension_semantics=(pltpu.ARBITRARY,),
        ),
    )(z, codebook)
    code_ids = ids3.reshape(b)

    z_q = _gather_rows(codebook, ids3.reshape(1, b), b, d)

    nl = b // _BL
    partials = pl.pallas_call(
        _loss_body,
        grid=(nl,),
        in_specs=[
            pl.BlockSpec((_BL, d), lambda i: (i, 0)),
            pl.BlockSpec((_BL, d), lambda i: (i, 0)),
        ],
        out_specs=pl.BlockSpec((1, 1, 1), lambda i: (i, 0, 0)),
        out_shape=jax.ShapeDtypeStruct((nl, 1, 1), jnp.float32),
        compiler_params=pltpu.CompilerParams(
            dimension_semantics=(pltpu.PARALLEL,),
        ),
    )(z, z_q)
    loss = (jnp.sum(partials) * (0.25 / (b * d))).astype(jnp.float32)

    return (z_q, code_ids, loss)


# speedup vs baseline: 1.0194x; 1.0104x over previous
import jax
import jax.numpy as jnp
from jax.experimental import pallas as pl
from jax.experimental.pallas import tpu as pltpu
from jax.experimental.pallas import tpu_sc as plsc

_BM = 1024
_KT = 1024
_BL = 4096


def _sim_argmax_body(z_ref, cb_ref, ids_ref, cbn_ref):
    i = pl.program_id(0)

    @pl.when(i == 0)
    def _normalize_codebook():
        cb = cb_ref[...]
        nrm = jnp.sqrt(jnp.sum(cb * cb, axis=1, keepdims=True))
        cbn_ref[...] = cb / jnp.maximum(nrm, 1e-12)

    z = z_ref[...]
    zn = z / jnp.maximum(jnp.sqrt(jnp.sum(z * z, axis=1, keepdims=True)), 1e-12)
    bm = z.shape[0]
    k = cb_ref.shape[0]
    kt = _KT
    run_max = jnp.full((bm, 128), -jnp.inf, jnp.float32)
    run_blk = jnp.zeros((bm, 128), jnp.int32)
    for t in range(k // kt):
        s = jax.lax.dot_general(
            zn, cbn_ref[pl.ds(t * kt, kt), :],
            dimension_numbers=(((1,), (1,)), ((), ())),
            preferred_element_type=jnp.float32,
        )
        for sub in range(kt // 128):
            x = s[:, sub * 128:(sub + 1) * 128]
            gt = x > run_max
            run_max = jnp.where(gt, x, run_max)
            run_blk = jnp.where(gt, t * (kt // 128) + sub, run_blk)
    j = jax.lax.broadcasted_iota(jnp.int32, (bm, 128), 1)
    kfull = run_blk * 128 + j
    maxv = jnp.max(run_max, axis=1, keepdims=True)
    ids_ref[0, 0, :] = jnp.min(jnp.where(run_max == maxv, kfull, k), axis=1)


def _loss_body(z_ref, zq_ref, out_ref):
    d = z_ref[...] - zq_ref[...]
    out_ref[...] = jnp.sum(d * d).reshape(1, 1, 1)


def _gather_rows(codebook, ids2d, n_rows, dim):
    mesh = plsc.VectorSubcoreMesh(core_axis_name="core", subcore_axis_name="subcore")
    window = 128

    @pl.kernel(
        out_type=jax.ShapeDtypeStruct((n_rows, dim), codebook.dtype),
        mesh=mesh,
    )
    def gather_kernel(cb_hbm, i_hbm, o_hbm):
        def body(i_vmem, o_vmem):
            pltpu.sync_copy(cb_hbm.at[i_vmem.at[0]], o_vmem)

        pltpu.emit_pipeline(
            body,
            grid=(n_rows // window,),
            in_specs=[pl.BlockSpec((1, window), lambda i: (0, i))],
            out_specs=[pl.BlockSpec((window, dim), lambda i: (i, 0))],
            core_axis_name=("core", "subcore"),
            dimension_semantics=(pltpu.PARALLEL,),
        )(i_hbm, o_hbm)

    return gather_kernel(codebook, ids2d)


def kernel(z, codebook):
    b, d = z.shape
    k, _ = codebook.shape
    nb = b // _BM

    ids3 = pl.pallas_call(
        _sim_argmax_body,
        grid=(nb,),
        in_specs=[
            pl.BlockSpec((_BM, d), lambda i: (i, 0)),
            pl.BlockSpec((k, d), lambda i: (0, 0)),
        ],
        out_specs=pl.BlockSpec((1, 1, _BM), lambda i: (i, 0, 0)),
        out_shape=jax.ShapeDtypeStruct((nb, 1, _BM), jnp.int32),
        scratch_shapes=[pltpu.VMEM((k, d), jnp.float32)],
        compiler_params=pltpu.CompilerParams(
            dimension_semantics=(pltpu.ARBITRARY,),
        ),
    )(z, codebook)
    code_ids = ids3.reshape(b)

    z_q = _gather_rows(codebook, ids3.reshape(1, b), b, d)

    nl = b // _BL
    partials = pl.pallas_call(
        _loss_body,
        grid=(nl,),
        in_specs=[
            pl.BlockSpec((_BL, d), lambda i: (i, 0)),
            pl.BlockSpec((_BL, d), lambda i: (i, 0)),
        ],
        out_specs=pl.BlockSpec((1, 1, 1), lambda i: (i, 0, 0)),
        out_shape=jax.ShapeDtypeStruct((nl, 1, 1), jnp.float32),
        compiler_params=pltpu.CompilerParams(
            dimension_semantics=(pltpu.PARALLEL,),
        ),
    )(z, z_q)
    loss = (jnp.sum(partials) * (0.25 / (b * d))).astype(jnp.float32)

    return (z_q, code_ids, loss)


# speedup vs baseline: 1.0197x; 1.0003x over previous
import jax
import jax.numpy as jnp
from jax.experimental import pallas as pl
from jax.experimental.pallas import tpu as pltpu
from jax.experimental.pallas import tpu_sc as plsc

_BM = 2048
_KT = 1024
_BL = 4096


def _sim_argmax_body(z_ref, cb_ref, ids_ref, cbn_ref):
    i = pl.program_id(0)

    @pl.when(i == 0)
    def _normalize_codebook():
        cb = cb_ref[...]
        nrm = jnp.sqrt(jnp.sum(cb * cb, axis=1, keepdims=True))
        cbn_ref[...] = cb / jnp.maximum(nrm, 1e-12)

    z = z_ref[...]
    zn = z / jnp.maximum(jnp.sqrt(jnp.sum(z * z, axis=1, keepdims=True)), 1e-12)
    bm = z.shape[0]
    k = cb_ref.shape[0]
    kt = _KT
    run_max = jnp.full((bm, 128), -jnp.inf, jnp.float32)
    run_blk = jnp.zeros((bm, 128), jnp.int32)
    for t in range(k // kt):
        s = jax.lax.dot_general(
            zn, cbn_ref[pl.ds(t * kt, kt), :],
            dimension_numbers=(((1,), (1,)), ((), ())),
            preferred_element_type=jnp.float32,
        )
        for sub in range(kt // 128):
            x = s[:, sub * 128:(sub + 1) * 128]
            gt = x > run_max
            run_max = jnp.where(gt, x, run_max)
            run_blk = jnp.where(gt, t * (kt // 128) + sub, run_blk)
    j = jax.lax.broadcasted_iota(jnp.int32, (bm, 128), 1)
    kfull = run_blk * 128 + j
    maxv = jnp.max(run_max, axis=1, keepdims=True)
    ids_ref[0, 0, :] = jnp.min(jnp.where(run_max == maxv, kfull, k), axis=1)


def _loss_body(z_ref, zq_ref, out_ref):
    d = z_ref[...] - zq_ref[...]
    out_ref[...] = jnp.sum(d * d).reshape(1, 1, 1)


def _gather_rows(codebook, ids2d, n_rows, dim):
    mesh = plsc.VectorSubcoreMesh(core_axis_name="core", subcore_axis_name="subcore")
    window = 128

    @pl.kernel(
        out_type=jax.ShapeDtypeStruct((n_rows, dim), codebook.dtype),
        mesh=mesh,
    )
    def gather_kernel(cb_hbm, i_hbm, o_hbm):
        def body(i_vmem, o_vmem):
            pltpu.sync_copy(cb_hbm.at[i_vmem.at[0]], o_vmem)

        pltpu.emit_pipeline(
            body,
            grid=(n_rows // window,),
            in_specs=[pl.BlockSpec((1, window), lambda i: (0, i))],
            out_specs=[pl.BlockSpec((window, dim), lambda i: (i, 0))],
            core_axis_name=("core", "subcore"),
            dimension_semantics=(pltpu.PARALLEL,),
        )(i_hbm, o_hbm)

    return gather_kernel(codebook, ids2d)


def kernel(z, codebook):
    b, d = z.shape
    k, _ = codebook.shape
    nb = b // _BM

    ids3 = pl.pallas_call(
        _sim_argmax_body,
        grid=(nb,),
        in_specs=[
            pl.BlockSpec((_BM, d), lambda i: (i, 0)),
            pl.BlockSpec((k, d), lambda i: (0, 0)),
        ],
        out_specs=pl.BlockSpec((1, 1, _BM), lambda i: (i, 0, 0)),
        out_shape=jax.ShapeDtypeStruct((nb, 1, _BM), jnp.int32),
        scratch_shapes=[pltpu.VMEM((k, d), jnp.float32)],
        compiler_params=pltpu.CompilerParams(
            dimension_semantics=(pltpu.ARBITRARY,),
        ),
    )(z, codebook)
    code_ids = ids3.reshape(b)

    z_q = _gather_rows(codebook, ids3.reshape(1, b), b, d)

    nl = b // _BL
    partials = pl.pallas_call(
        _loss_body,
        grid=(nl,),
        in_specs=[
            pl.BlockSpec((_BL, d), lambda i: (i, 0)),
            pl.BlockSpec((_BL, d), lambda i: (i, 0)),
        ],
        out_specs=pl.BlockSpec((1, 1, 1), lambda i: (i, 0, 0)),
        out_shape=jax.ShapeDtypeStruct((nl, 1, 1), jnp.float32),
        compiler_params=pltpu.CompilerParams(
            dimension_semantics=(pltpu.PARALLEL,),
        ),
    )(z, z_q)
    loss = (jnp.sum(partials) * (0.25 / (b * d))).astype(jnp.float32)

    return (z_q, code_ids, loss)
